# Initial kernel scaffold; baseline (speedup 1.0000x reference)
import jax
import jax.numpy as jnp
from jax.experimental import pallas as pl


def kernel(seq_ids, seq_len, table):
    raise NotImplementedError("write your pallas kernel here")


# speedup vs baseline: 2.8841x; 2.8841x over previous
import functools

import jax
import jax.numpy as jnp
from jax import lax
from jax.experimental import pallas as pl
from jax.experimental.pallas import tpu as pltpu
from jax.experimental.pallas import tpu_sc as plsc

B = 4096
L = 50
LP = 56
CH = 8
D = 768
LANES = 16
NV = D // LANES
NW = 32
BW = B // NW
GRP = 16


def _sc_body(ids_hbm, len_hbm, table_hbm, out_hbm,
             ids_v, len_v, rows_v, oslab, sem0, sem1):
  cid = lax.axis_index("c")
  sid = lax.axis_index("s")
  wid = sid * 2 + cid
  base = pl.multiple_of(wid * BW, BW)

  pltpu.sync_copy(ids_hbm.at[pl.ds(base, BW)], ids_v)
  pltpu.sync_copy(len_hbm.at[pl.ds(base, BW + LANES)], len_v)

  iota = lax.iota(jnp.int32, LANES)
  zero = jnp.zeros((LANES,), jnp.float32)

  def row_len(b):
    l16 = len_v[pl.ds(b, LANES)]
    return l16[0]

  def n_chunks(lenb):
    return (lenb + CH - 1) // CH

  def issue(b, buf, sem):
    lenb = row_len(b)

    def ibody(ci, carry):
      pltpu.async_copy(
          table_hbm.at[ids_v.at[b, pl.ds(ci * CH, CH)]],
          rows_v.at[buf].at[pl.ds(ci * CH, CH)],
          sem)
      return carry

    lax.fori_loop(0, n_chunks(lenb), ibody, 0)

  def drain(buf, sem, nc):
    def dbody(ci, carry):
      pltpu.make_async_copy(
          table_hbm.at[pl.ds(0, CH)],
          rows_v.at[buf].at[pl.ds(0, CH)],
          sem).wait()
      return carry

    lax.fori_loop(0, nc, dbody, 0)

  def process(b, buf, sem):
    lenb = row_len(b)
    drain(buf, sem, n_chunks(lenb))

    i16 = b % GRP
    for k in range(NV):
      oslab[i16, pl.ds(LANES * k, LANES)] = zero

    def abody(p, carry):
      for k in range(NV):
        plsc.addupdate(oslab.at[i16, pl.ds(LANES * k, LANES)],
                       rows_v[buf, p, pl.ds(LANES * k, LANES)])
      return carry

    lax.fori_loop(0, lenb, abody, 0)

    lsplat = jnp.full((LANES,), jnp.maximum(lenb, 1), jnp.int32)
    invv = jnp.float32(1.0) / lsplat.astype(jnp.float32)
    for k in range(NV):
      sl = pl.ds(LANES * k, LANES)
      oslab[i16, sl] = oslab[i16, sl] * invv

    @pl.when(i16 == GRP - 1)
    def _flush():
      off = pl.multiple_of(base + b - (GRP - 1), GRP)
      pltpu.sync_copy(oslab, out_hbm.at[pl.ds(off, GRP)])

  issue(0, 0, sem0)
  issue(1, 1, sem1)

  def pair(g, carry):
    b0 = 2 * g
    process(b0, 0, sem0)

    @pl.when(b0 + 2 < BW)
    def _():
      issue(b0 + 2, 0, sem0)

    process(b0 + 1, 1, sem1)

    @pl.when(b0 + 3 < BW)
    def _():
      issue(b0 + 3, 1, sem1)

    return carry

  lax.fori_loop(0, BW // 2, pair, 0)


@jax.jit
def _run(ids_pad, seq_len, table):
  mesh = plsc.VectorSubcoreMesh(core_axis_name="c", subcore_axis_name="s")
  f = pl.kernel(
      _sc_body,
      out_type=jax.ShapeDtypeStruct((B, D), jnp.float32),
      mesh=mesh,
      scratch_types=[
          pltpu.VMEM((BW, LP), jnp.int32),
          pltpu.VMEM((BW + LANES,), jnp.int32),
          pltpu.VMEM((2, LP, D), jnp.float32),
          pltpu.VMEM((GRP, D), jnp.float32),
          pltpu.SemaphoreType.DMA,
          pltpu.SemaphoreType.DMA,
      ],
  )
  return f(ids_pad, seq_len, table)


def kernel(seq_ids, seq_len, table):
  ids_pad = jnp.pad(seq_ids.astype(jnp.int32), ((0, 0), (0, LP - L)))
  len_pad = jnp.pad(seq_len.astype(jnp.int32), (0, LANES), constant_values=1)
  return _run(ids_pad, len_pad, table)
